# baseline (speedup 1.0000x reference)
import functools

import jax
import jax.numpy as jnp
from jax import lax
from jax.experimental import pallas as pl
from jax.experimental.pallas import tpu as pltpu
from jax.experimental.pallas import tpu_sc as plsc

_NUM_LEAD_TIMES = 722
_EMBED_DIM = 32
_BATCH = 4096

_INFO = plsc.get_sparse_core_info()
_NS = _INFO.num_subcores
_NW = _NS
_B_PER_W = _BATCH // _NW
_CHUNK = 128


@functools.partial(
    pl.kernel,
    mesh=plsc.VectorSubcoreMesh(core_axis_name="c", subcore_axis_name="s",
                                num_cores=1),
    out_type=jax.ShapeDtypeStruct((_BATCH, _EMBED_DIM), jnp.float32),
    scratch_types=[
        pltpu.VMEM((_B_PER_W,), jnp.int32),
        pltpu.VMEM((_CHUNK, _EMBED_DIM), jnp.float32),
        pltpu.VMEM((_CHUNK, _EMBED_DIM), jnp.float32),
        pltpu.SemaphoreType.DMA,
        pltpu.SemaphoreType.DMA,
        pltpu.SemaphoreType.DMA,
        pltpu.SemaphoreType.DMA,
        pltpu.SemaphoreType.DMA,
    ],
    compiler_params=pltpu.CompilerParams(use_tc_tiling_on_sc=False),
)
def _sc_gather(table_hbm, idx_hbm, out_hbm, idx_v, rows0_v, rows1_v,
               sem_i, sem0, sem1, sem_w0, sem_w1):
    wid = lax.axis_index("s")
    base = wid * _B_PER_W
    pltpu.sync_copy(idx_hbm.at[pl.ds(base, _CHUNK)], idx_v.at[pl.ds(0, _CHUNK)])
    g0 = pltpu.async_copy(table_hbm.at[idx_v.at[pl.ds(0, _CHUNK)]],
                          rows0_v, sem0)
    i1 = pltpu.async_copy(idx_hbm.at[pl.ds(base + _CHUNK, _CHUNK)],
                          idx_v.at[pl.ds(_CHUNK, _CHUNK)], sem_i)
    i1.wait()
    g1 = pltpu.async_copy(table_hbm.at[idx_v.at[pl.ds(_CHUNK, _CHUNK)]],
                          rows1_v, sem1)
    g0.wait()
    w0 = pltpu.async_copy(rows0_v, out_hbm.at[pl.ds(base, _CHUNK)], sem_w0)
    g1.wait()
    w1 = pltpu.async_copy(rows1_v, out_hbm.at[pl.ds(base + _CHUNK, _CHUNK)],
                          sem_w1)
    w0.wait()
    w1.wait()


def kernel(lead_times, sparse_inputs, dense_inputs_2496, dense_inputs_4996,
           lead_time_embedding):
    del sparse_inputs, dense_inputs_2496, dense_inputs_4996
    return _sc_gather(lead_time_embedding, lead_times.astype(jnp.int32))


# speedup vs baseline: 1.0082x; 1.0082x over previous
import functools

import jax
import jax.numpy as jnp
from jax import lax
from jax.experimental import pallas as pl
from jax.experimental.pallas import tpu as pltpu
from jax.experimental.pallas import tpu_sc as plsc

_NUM_LEAD_TIMES = 722
_EMBED_DIM = 32
_BATCH = 4096

_INFO = plsc.get_sparse_core_info()
_NS = _INFO.num_subcores
_NW = _NS
_B_PER_W = _BATCH // _NW
_CHUNK = 128


@functools.partial(
    pl.kernel,
    mesh=plsc.VectorSubcoreMesh(core_axis_name="c", subcore_axis_name="s",
                                num_cores=1),
    out_type=jax.ShapeDtypeStruct((_BATCH, _EMBED_DIM), jnp.float32),
    scratch_types=[
        pltpu.VMEM((_B_PER_W,), jnp.int32),
        pltpu.VMEM((_CHUNK, _EMBED_DIM), jnp.float32),
        pltpu.VMEM((_CHUNK, _EMBED_DIM), jnp.float32),
        pltpu.SemaphoreType.DMA,
        pltpu.SemaphoreType.DMA,
        pltpu.SemaphoreType.DMA,
        pltpu.SemaphoreType.DMA,
    ],
    compiler_params=pltpu.CompilerParams(use_tc_tiling_on_sc=False),
)
def _sc_gather(table_hbm, idx_hbm, out_hbm, idx_v, rows0_v, rows1_v,
               sem0, sem1, sem_w0, sem_w1):
    wid = lax.axis_index("s")
    base = wid * _B_PER_W
    pltpu.sync_copy(idx_hbm.at[pl.ds(base, _B_PER_W)], idx_v)
    g0 = pltpu.async_copy(table_hbm.at[idx_v.at[pl.ds(0, _CHUNK)]],
                          rows0_v, sem0)
    g1 = pltpu.async_copy(table_hbm.at[idx_v.at[pl.ds(_CHUNK, _CHUNK)]],
                          rows1_v, sem1)
    g0.wait()
    w0 = pltpu.async_copy(rows0_v, out_hbm.at[pl.ds(base, _CHUNK)], sem_w0)
    g1.wait()
    w1 = pltpu.async_copy(rows1_v, out_hbm.at[pl.ds(base + _CHUNK, _CHUNK)],
                          sem_w1)
    w0.wait()
    w1.wait()


def kernel(lead_times, sparse_inputs, dense_inputs_2496, dense_inputs_4996,
           lead_time_embedding):
    del sparse_inputs, dense_inputs_2496, dense_inputs_4996
    return _sc_gather(lead_time_embedding, lead_times.astype(jnp.int32))
